# Initial kernel scaffold; baseline (speedup 1.0000x reference)
import functools

import jax
import jax.numpy as jnp
from jax import lax
from jax.experimental import pallas as pl
from jax.experimental.pallas import tpu as pltpu
from jax.experimental.pallas import tpu_sc as plsc

N = 10000
E = 160000
N_PAD = 10240
NC = 2
NS = 16
DUMMY = N
ZR = 160
RPT = N_PAD // NS

SS_NB = 79
SS_EPT = SS_NB * 128
DG_NB = 40
DG_EPW = DG_NB * 128

BLK = 512


def _deg_kernel(dst_w, ones_hbm, zeros16_hbm):
  mesh = plsc.VectorSubcoreMesh(core_axis_name="c", subcore_axis_name="s")

  @functools.partial(
      pl.kernel,
      mesh=mesh,
      out_type=jax.ShapeDtypeStruct((NC, N_PAD, 16), jnp.float32),
      scratch_types=[
          pltpu.VMEM((DG_NB, 128), jnp.int32),
          pltpu.VMEM((128, 16), jnp.float32),
          pltpu.VMEM((RPT, 16), jnp.float32),
          pltpu.VMEM_SHARED((N_PAD, 16), jnp.float32),
      ],
  )
  def k(dst_hbm, ones_h, zeros_h, out_hbm, dst_v, ones_v, stage_v, acc_sh):
    c = lax.axis_index("c")
    s = lax.axis_index("s")
    pltpu.sync_copy(dst_hbm.at[c, s], dst_v)
    pltpu.sync_copy(ones_h, ones_v)
    pltpu.sync_copy(zeros_h, stage_v)
    pltpu.sync_copy(stage_v, acc_sh.at[pl.ds(s * RPT, RPT)])
    plsc.subcore_barrier()
    def body(j, carry):
      pltpu.sync_copy(ones_v, acc_sh.at[dst_v.at[j]], add=True)
      return carry
    lax.fori_loop(0, DG_NB, body, 0)
    plsc.subcore_barrier()
    pltpu.sync_copy(acc_sh.at[pl.ds(s * RPT, RPT)], stage_v)
    pltpu.sync_copy(stage_v, out_hbm.at[c, pl.ds(s * RPT, RPT)])

  return k(dst_w, ones_hbm, zeros16_hbm)


def _segsum_kernel(hs_flat, src_w, dst_w, zeros_hbm, C):
  c_half = C // NC
  mesh = plsc.VectorSubcoreMesh(core_axis_name="c", subcore_axis_name="s")

  @functools.partial(
      pl.kernel,
      mesh=mesh,
      out_type=jax.ShapeDtypeStruct((C * N_PAD, 128), jnp.float32),
      scratch_types=[
          pltpu.VMEM((SS_NB, 128), jnp.int32),
          pltpu.VMEM((SS_NB, 128), jnp.int32),
          pltpu.VMEM((128, 128), jnp.float32),
          pltpu.VMEM((ZR, 128), jnp.float32),
          pltpu.VMEM((ZR, 128), jnp.float32),
          pltpu.VMEM_SHARED((N_PAD, 128), jnp.float32),
          pltpu.SemaphoreType.DMA,
      ],
  )
  def k(hs_hbm, src_hbm, dst_hbm, z_hbm, out_hbm,
        src_v, dst_v, rowbuf, zbuf, obuf, acc_sh, sem):
    c = lax.axis_index("c")
    s = lax.axis_index("s")
    pltpu.sync_copy(z_hbm, zbuf)
    pltpu.sync_copy(dst_hbm.at[s], dst_v)
    base = s * RPT
    for ci in range(c_half):
      chunk = ci * NC + c
      pltpu.sync_copy(src_hbm.at[chunk, s], src_v)
      for z in range(RPT // ZR):
        pltpu.sync_copy(zbuf, acc_sh.at[pl.ds(base + z * ZR, ZR)])
      plsc.subcore_barrier()

      def body(j, carry):
        pltpu.async_copy(hs_hbm.at[src_v.at[j]], rowbuf, sem).wait()
        pltpu.sync_copy(rowbuf, acc_sh.at[dst_v.at[j]], add=True)
        return carry
      lax.fori_loop(0, SS_NB, body, 0)
      plsc.subcore_barrier()

      out_base = chunk * N_PAD + base
      for z in range(RPT // ZR):
        pltpu.sync_copy(acc_sh.at[pl.ds(base + z * ZR, ZR)], obuf)
        pltpu.sync_copy(obuf, out_hbm.at[pl.ds(out_base + z * ZR, ZR)])
      plsc.subcore_barrier()

  return k(hs_flat, src_w, dst_w, zeros_hbm)


def _prep_edges(a):
  src = a[0]
  dst = a[1]
  pad_ss = NS * SS_EPT - E
  src_ss = jnp.concatenate([src, jnp.zeros((pad_ss,), jnp.int32)])
  dst_ss = jnp.concatenate([dst, jnp.full((pad_ss,), DUMMY, jnp.int32)])
  src_ss = src_ss.reshape(NS, SS_NB, 128)
  dst_ss = dst_ss.reshape(NS, SS_NB, 128)
  off4 = (jnp.arange(4, dtype=jnp.int32) * N_PAD)[:, None, None, None]
  src4 = src_ss[None] + off4
  src2 = src4[:2]
  pad_dg = NC * NS * DG_EPW - E
  dst_dg = jnp.concatenate([dst, jnp.full((pad_dg,), DUMMY, jnp.int32)])
  dst_dg = dst_dg.reshape(NC, NS, DG_NB, 128)
  return src4, src2, dst_ss, dst_dg


def _dis_from_pdeg(pdeg_blk):
  deg = pdeg_blk[0, :, 0:1] + pdeg_blk[1, :, 0:1] + 1.0
  return lax.rsqrt(deg)


def _rep(shape):
  return pl.BlockSpec(shape, lambda i: (0,) * len(shape))


def _rows(shape):
  return pl.BlockSpec(shape, lambda i: (i,) + (0,) * (len(shape) - 1))


def _pdeg_spec():
  return pl.BlockSpec((NC, BLK, 16), lambda i: (0, i, 0))


def _chunked_out(c):
  return pl.BlockSpec((c, BLK, 128), lambda i: (0, i, 0))


def _to_chunks(h, c):
  return jnp.transpose(h.reshape(BLK, c, 128), (1, 0, 2))


def _from_chunks(hc, c):
  return jnp.transpose(hc, (1, 0, 2)).reshape(BLK, c * 128)


def _tc1(x_pad, W0, pdeg):
  def body(x_ref, w_ref, pd_ref, hh_ref, hs_ref):
    dis = _dis_from_pdeg(pd_ref[...])
    hh = jnp.dot(x_ref[...], w_ref[...], preferred_element_type=jnp.float32)
    hh_ref[...] = hh
    hs_ref[...] = _to_chunks(dis * hh, 4)

  return pl.pallas_call(
      body,
      grid=(N_PAD // BLK,),
      in_specs=[_rows((BLK, 256)), _rep((256, 512)), _pdeg_spec()],
      out_specs=[_rows((BLK, 512)), _chunked_out(4)],
      out_shape=[
          jax.ShapeDtypeStruct((N_PAD, 512), jnp.float32),
          jax.ShapeDtypeStruct((4, N_PAD, 128), jnp.float32),
      ],
  )(x_pad, W0, pdeg)


def _tc2(agg0, hh0, pdeg, b0, alpha, W1, Ws0, bs0):
  def body(ag_ref, hh_ref, pd_ref, b0_ref, al_ref, w1_ref, ws_ref, bs_ref,
           hh1_ref, hs1_ref, rs_ref):
    dis = _dis_from_pdeg(pd_ref[...])
    agg = _from_chunks(ag_ref[...], 4)
    hh0 = hh_ref[...]
    z = dis * agg + (dis * dis) * hh0 + b0_ref[...]
    al = al_ref[0]
    z = jnp.where(z >= 0, z, al * z)
    hh1 = jnp.dot(z, w1_ref[...], preferred_element_type=jnp.float32)
    hh1_ref[...] = hh1
    hs1_ref[...] = _to_chunks(dis * hh1, 4)
    rs_ref[...] = jnp.dot(z, ws_ref[...],
                          preferred_element_type=jnp.float32) + bs_ref[...]

  return pl.pallas_call(
      body,
      grid=(N_PAD // BLK,),
      in_specs=[
          pl.BlockSpec((4, BLK, 128), lambda i: (0, i, 0)),
          _rows((BLK, 512)), _pdeg_spec(), _rep((1, 512)),
          pl.BlockSpec(memory_space=pltpu.MemorySpace.SMEM),
          _rep((512, 512)), _rep((512, 512)), _rep((1, 512)),
      ],
      out_specs=[_rows((BLK, 512)), _chunked_out(4), _rows((BLK, 512))],
      out_shape=[
          jax.ShapeDtypeStruct((N_PAD, 512), jnp.float32),
          jax.ShapeDtypeStruct((4, N_PAD, 128), jnp.float32),
          jax.ShapeDtypeStruct((N_PAD, 512), jnp.float32),
      ],
  )(agg0, hh0, pdeg, b0, alpha, W1, Ws0, bs0)


def _tc3(agg1, hh1, resid1, pdeg, b1, W2, Ws1, bs1):
  def body(ag_ref, hh_ref, r_ref, pd_ref, b1_ref, w2_ref, ws_ref, bs_ref,
           hh2_ref, hs2_ref, rs_ref):
    dis = _dis_from_pdeg(pd_ref[...])
    agg = _from_chunks(ag_ref[...], 4)
    z = dis * agg + (dis * dis) * hh_ref[...] + b1_ref[...] + r_ref[...]
    hh2 = jnp.dot(z, w2_ref[...], preferred_element_type=jnp.float32)
    hh2_ref[...] = hh2
    hs2_ref[...] = _to_chunks(dis * hh2, 2)
    rs_ref[...] = jnp.dot(z, ws_ref[...],
                          preferred_element_type=jnp.float32) + bs_ref[...]

  return pl.pallas_call(
      body,
      grid=(N_PAD // BLK,),
      in_specs=[
          pl.BlockSpec((4, BLK, 128), lambda i: (0, i, 0)),
          _rows((BLK, 512)), _rows((BLK, 512)), _pdeg_spec(), _rep((1, 512)),
          _rep((512, 256)), _rep((512, 256)), _rep((1, 256)),
      ],
      out_specs=[_rows((BLK, 256)), _chunked_out(2), _rows((BLK, 256))],
      out_shape=[
          jax.ShapeDtypeStruct((N_PAD, 256), jnp.float32),
          jax.ShapeDtypeStruct((2, N_PAD, 128), jnp.float32),
          jax.ShapeDtypeStruct((N_PAD, 256), jnp.float32),
      ],
  )(agg1, hh1, resid1, pdeg, b1, W2, Ws1, bs1)


def _tc4(agg2, hh2, resid2, pdeg, b2, Wp0, bp0, Wp1, bp1):
  def body(ag_ref, hh_ref, r_ref, pd_ref, b2_ref, w0_ref, c0_ref,
           w1_ref, c1_ref, out_ref):
    dis = _dis_from_pdeg(pd_ref[...])
    agg = _from_chunks(ag_ref[...], 2)
    z = dis * agg + (dis * dis) * hh_ref[...] + b2_ref[...] + r_ref[...]
    t = jnp.dot(z, w0_ref[...], preferred_element_type=jnp.float32)
    t = t + c0_ref[...]
    o = jnp.dot(t, w1_ref[...], preferred_element_type=jnp.float32)
    out_ref[...] = o + c1_ref[...]

  return pl.pallas_call(
      body,
      grid=(N_PAD // BLK,),
      in_specs=[
          pl.BlockSpec((2, BLK, 128), lambda i: (0, i, 0)),
          _rows((BLK, 256)), _rows((BLK, 256)), _pdeg_spec(), _rep((1, 256)),
          _rep((256, 256)), _rep((1, 256)), _rep((256, 128)), _rep((1, 128)),
      ],
      out_specs=[_rows((BLK, 128))],
      out_shape=[jax.ShapeDtypeStruct((N_PAD, 128), jnp.float32)],
  )(agg2, hh2, resid2, pdeg, b2, Wp0, bp0, Wp1, bp1)[0]


@jax.jit
def kernel(x, a, p, W0, b0, W1, b1, W2, b2, Ws0, bs0, Ws1, bs1,
           Wp0, bp0, Wp1, bp1, alpha):
  del p
  src4, src2, dst_ss, dst_dg = _prep_edges(a)
  ones16 = jnp.ones((128, 16), jnp.float32)
  zeros16 = jnp.zeros((RPT, 16), jnp.float32)
  zeros128 = jnp.zeros((ZR, 128), jnp.float32)
  x_pad = jnp.pad(x, ((0, N_PAD - N), (0, 0)))
  alpha1 = alpha.reshape(1)

  pdeg = _deg_kernel(dst_dg, ones16, zeros16)

  hh0, hs0 = _tc1(x_pad, W0, pdeg)
  agg0 = _segsum_kernel(hs0.reshape(4 * N_PAD, 128), src4, dst_ss,
                        zeros128, 4).reshape(4, N_PAD, 128)

  hh1, hs1, resid1 = _tc2(agg0, hh0, pdeg, b0.reshape(1, 512), alpha1,
                          W1, Ws0, bs0.reshape(1, 512))
  agg1 = _segsum_kernel(hs1.reshape(4 * N_PAD, 128), src4, dst_ss,
                        zeros128, 4).reshape(4, N_PAD, 128)

  hh2, hs2, resid2 = _tc3(agg1, hh1, resid1, pdeg, b1.reshape(1, 512),
                          W2, Ws1, bs1.reshape(1, 256))
  agg2 = _segsum_kernel(hs2.reshape(2 * N_PAD, 128), src2, dst_ss,
                        zeros128, 2).reshape(2, N_PAD, 128)

  out = _tc4(agg2, hh2, resid2, pdeg, b2.reshape(1, 256),
             Wp0, bp0.reshape(1, 256), Wp1, bp1.reshape(1, 128))
  return out[:N]


# speedup vs baseline: 5.2476x; 5.2476x over previous
import functools

import jax
import jax.numpy as jnp
from jax import lax
from jax.experimental import pallas as pl
from jax.experimental.pallas import tpu as pltpu
from jax.experimental.pallas import tpu_sc as plsc

N = 10000
E = 160000
N_PAD = 10240
NC = 2
NS = 16
DUMMY = N
ZR = 160
RPT = N_PAD // NS

SS_NB = 80
SS_EPT = SS_NB * 128
DG_NB = 40
DG_EPW = DG_NB * 128

BLK = 512


def _deg_kernel(dst_w, ones_hbm, zeros_hbm):
  mesh = plsc.VectorSubcoreMesh(core_axis_name="c", subcore_axis_name="s")

  @functools.partial(
      pl.kernel,
      mesh=mesh,
      out_type=jax.ShapeDtypeStruct((NC * N_PAD, 128), jnp.float32),
      scratch_types=[
          pltpu.VMEM((DG_NB, 128), jnp.int32),
          pltpu.VMEM((128, 128), jnp.float32),
          pltpu.VMEM((128, 128), jnp.float32),
          pltpu.VMEM_SHARED((N_PAD, 128), jnp.float32),
      ],
  )
  def k(dst_hbm, ones_h, z_hbm, out_hbm, dst_v, ones_v, stage, acc_sh):
    c = lax.axis_index("c")
    s = lax.axis_index("s")
    pltpu.sync_copy(dst_hbm.at[c, s], dst_v)
    pltpu.sync_copy(ones_h, ones_v)
    base = s * RPT
    pltpu.sync_copy(z_hbm, stage)
    for z in range(RPT // 128):
      pltpu.sync_copy(stage, acc_sh.at[pl.ds(base + z * 128, 128)])
    plsc.subcore_barrier()

    def body(j, carry):
      pltpu.sync_copy(ones_v, acc_sh.at[dst_v.at[j]], add=True)
      return carry
    lax.fori_loop(0, DG_NB, body, 0)
    plsc.subcore_barrier()

    out_base = c * N_PAD + base
    for z in range(RPT // 128):
      pltpu.sync_copy(acc_sh.at[pl.ds(base + z * 128, 128)], stage)
      pltpu.sync_copy(stage, out_hbm.at[pl.ds(out_base + z * 128, 128)])

  return k(dst_w, ones_hbm, zeros_hbm)


def _segsum_kernel(hs_flat, src_w, dst_w, zeros_hbm, C):
  c_half = C // NC
  mesh = plsc.VectorSubcoreMesh(core_axis_name="c", subcore_axis_name="s")

  @functools.partial(
      pl.kernel,
      mesh=mesh,
      out_type=jax.ShapeDtypeStruct((C * N_PAD, 128), jnp.float32),
      scratch_types=[
          pltpu.VMEM((SS_NB, 128), jnp.int32),
          pltpu.VMEM((SS_NB, 128), jnp.int32),
          pltpu.VMEM((128, 128), jnp.float32),
          pltpu.VMEM_SHARED((N_PAD, 128), jnp.float32),
          pltpu.SemaphoreType.DMA,
      ],
  )
  def k(hs_hbm, src_hbm, dst_hbm, z_hbm, out_hbm,
        src_v, dst_v, rowbuf, acc_sh, sem):
    c = lax.axis_index("c")
    s = lax.axis_index("s")
    pltpu.sync_copy(dst_hbm.at[s], dst_v)
    base = s * RPT
    for ci in range(c_half):
      chunk = ci * NC + c
      pltpu.sync_copy(src_hbm.at[chunk, s], src_v)
      pltpu.sync_copy(z_hbm, rowbuf)
      for z in range(RPT // 128):
        pltpu.sync_copy(rowbuf, acc_sh.at[pl.ds(base + z * 128, 128)])
      plsc.subcore_barrier()

      def body(j, carry):
        pltpu.async_copy(hs_hbm.at[src_v.at[j]], rowbuf, sem).wait()
        pltpu.sync_copy(rowbuf, acc_sh.at[dst_v.at[j]], add=True)
        return carry
      lax.fori_loop(0, SS_NB, body, 0)
      plsc.subcore_barrier()

      out_base = chunk * N_PAD + base
      for z in range(RPT // 128):
        pltpu.sync_copy(acc_sh.at[pl.ds(base + z * 128, 128)], rowbuf)
        pltpu.sync_copy(rowbuf, out_hbm.at[pl.ds(out_base + z * 128, 128)])
      plsc.subcore_barrier()

  return k(hs_flat, src_w, dst_w, zeros_hbm)


def _prep_edges(a):
  src = a[0]
  dst = a[1]
  pad_ss = NS * SS_EPT - E
  src_ss = jnp.concatenate([src, jnp.zeros((pad_ss,), jnp.int32)])
  dst_ss = jnp.concatenate([dst, jnp.full((pad_ss,), DUMMY, jnp.int32)])
  src_ss = src_ss.reshape(NS, SS_NB, 128)
  dst_ss = dst_ss.reshape(NS, SS_NB, 128)
  off4 = (jnp.arange(4, dtype=jnp.int32) * N_PAD)[:, None, None, None]
  src4 = src_ss[None] + off4
  src2 = src4[:2]
  pad_dg = NC * NS * DG_EPW - E
  dst_dg = jnp.concatenate([dst, jnp.full((pad_dg,), DUMMY, jnp.int32)])
  dst_dg = dst_dg.reshape(NC, NS, DG_NB, 128)
  return src4, src2, dst_ss, dst_dg


def _dis_from_pdeg(pdeg_blk):
  deg = pdeg_blk[0, :, 0:1] + pdeg_blk[1, :, 0:1] + 1.0
  return lax.rsqrt(deg)


def _rep(shape):
  return pl.BlockSpec(shape, lambda i: (0,) * len(shape))


def _rows(shape):
  return pl.BlockSpec(shape, lambda i: (i,) + (0,) * (len(shape) - 1))


def _pdeg_spec():
  return pl.BlockSpec((NC, BLK, 128), lambda i: (0, i, 0))


def _chunked_out(c):
  return pl.BlockSpec((c, BLK, 128), lambda i: (0, i, 0))


def _to_chunks(h, c):
  return jnp.transpose(h.reshape(BLK, c, 128), (1, 0, 2))


def _from_chunks(hc, c):
  return jnp.transpose(hc, (1, 0, 2)).reshape(BLK, c * 128)


def _tc1(x_pad, W0, pdeg):
  def body(x_ref, w_ref, pd_ref, hh_ref, hs_ref):
    dis = _dis_from_pdeg(pd_ref[...])
    hh = jnp.dot(x_ref[...], w_ref[...], preferred_element_type=jnp.float32)
    hh_ref[...] = hh
    hs_ref[...] = _to_chunks(dis * hh, 4)

  return pl.pallas_call(
      body,
      grid=(N_PAD // BLK,),
      in_specs=[_rows((BLK, 256)), _rep((256, 512)), _pdeg_spec()],
      out_specs=[_rows((BLK, 512)), _chunked_out(4)],
      out_shape=[
          jax.ShapeDtypeStruct((N_PAD, 512), jnp.float32),
          jax.ShapeDtypeStruct((4, N_PAD, 128), jnp.float32),
      ],
  )(x_pad, W0, pdeg)


def _tc2(agg0, hh0, pdeg, b0, alpha, W1, Ws0, bs0):
  def body(ag_ref, hh_ref, pd_ref, b0_ref, al_ref, w1_ref, ws_ref, bs_ref,
           hh1_ref, hs1_ref, rs_ref):
    dis = _dis_from_pdeg(pd_ref[...])
    agg = _from_chunks(ag_ref[...], 4)
    hh0 = hh_ref[...]
    z = dis * agg + (dis * dis) * hh0 + b0_ref[...]
    al = al_ref[0]
    z = jnp.where(z >= 0, z, al * z)
    hh1 = jnp.dot(z, w1_ref[...], preferred_element_type=jnp.float32)
    hh1_ref[...] = hh1
    hs1_ref[...] = _to_chunks(dis * hh1, 4)
    rs_ref[...] = jnp.dot(z, ws_ref[...],
                          preferred_element_type=jnp.float32) + bs_ref[...]

  return pl.pallas_call(
      body,
      grid=(N_PAD // BLK,),
      in_specs=[
          pl.BlockSpec((4, BLK, 128), lambda i: (0, i, 0)),
          _rows((BLK, 512)), _pdeg_spec(), _rep((1, 512)),
          pl.BlockSpec(memory_space=pltpu.MemorySpace.SMEM),
          _rep((512, 512)), _rep((512, 512)), _rep((1, 512)),
      ],
      out_specs=[_rows((BLK, 512)), _chunked_out(4), _rows((BLK, 512))],
      out_shape=[
          jax.ShapeDtypeStruct((N_PAD, 512), jnp.float32),
          jax.ShapeDtypeStruct((4, N_PAD, 128), jnp.float32),
          jax.ShapeDtypeStruct((N_PAD, 512), jnp.float32),
      ],
  )(agg0, hh0, pdeg, b0, alpha, W1, Ws0, bs0)


def _tc3(agg1, hh1, resid1, pdeg, b1, W2, Ws1, bs1):
  def body(ag_ref, hh_ref, r_ref, pd_ref, b1_ref, w2_ref, ws_ref, bs_ref,
           hh2_ref, hs2_ref, rs_ref):
    dis = _dis_from_pdeg(pd_ref[...])
    agg = _from_chunks(ag_ref[...], 4)
    z = dis * agg + (dis * dis) * hh_ref[...] + b1_ref[...] + r_ref[...]
    hh2 = jnp.dot(z, w2_ref[...], preferred_element_type=jnp.float32)
    hh2_ref[...] = hh2
    hs2_ref[...] = _to_chunks(dis * hh2, 2)
    rs_ref[...] = jnp.dot(z, ws_ref[...],
                          preferred_element_type=jnp.float32) + bs_ref[...]

  return pl.pallas_call(
      body,
      grid=(N_PAD // BLK,),
      in_specs=[
          pl.BlockSpec((4, BLK, 128), lambda i: (0, i, 0)),
          _rows((BLK, 512)), _rows((BLK, 512)), _pdeg_spec(), _rep((1, 512)),
          _rep((512, 256)), _rep((512, 256)), _rep((1, 256)),
      ],
      out_specs=[_rows((BLK, 256)), _chunked_out(2), _rows((BLK, 256))],
      out_shape=[
          jax.ShapeDtypeStruct((N_PAD, 256), jnp.float32),
          jax.ShapeDtypeStruct((2, N_PAD, 128), jnp.float32),
          jax.ShapeDtypeStruct((N_PAD, 256), jnp.float32),
      ],
  )(agg1, hh1, resid1, pdeg, b1, W2, Ws1, bs1)


def _tc4(agg2, hh2, resid2, pdeg, b2, Wp0, bp0, Wp1, bp1):
  def body(ag_ref, hh_ref, r_ref, pd_ref, b2_ref, w0_ref, c0_ref,
           w1_ref, c1_ref, out_ref):
    dis = _dis_from_pdeg(pd_ref[...])
    agg = _from_chunks(ag_ref[...], 2)
    z = dis * agg + (dis * dis) * hh_ref[...] + b2_ref[...] + r_ref[...]
    t = jnp.dot(z, w0_ref[...], preferred_element_type=jnp.float32)
    t = t + c0_ref[...]
    o = jnp.dot(t, w1_ref[...], preferred_element_type=jnp.float32)
    out_ref[...] = o + c1_ref[...]

  return pl.pallas_call(
      body,
      grid=(N_PAD // BLK,),
      in_specs=[
          pl.BlockSpec((2, BLK, 128), lambda i: (0, i, 0)),
          _rows((BLK, 256)), _rows((BLK, 256)), _pdeg_spec(), _rep((1, 256)),
          _rep((256, 256)), _rep((1, 256)), _rep((256, 128)), _rep((1, 128)),
      ],
      out_specs=[_rows((BLK, 128))],
      out_shape=[jax.ShapeDtypeStruct((N_PAD, 128), jnp.float32)],
  )(agg2, hh2, resid2, pdeg, b2, Wp0, bp0, Wp1, bp1)[0]


@jax.jit
def kernel(x, a, p, W0, b0, W1, b1, W2, b2, Ws0, bs0, Ws1, bs1,
           Wp0, bp0, Wp1, bp1, alpha):
  del p
  src4, src2, dst_ss, dst_dg = _prep_edges(a)
  ones128 = jnp.ones((128, 128), jnp.float32)
  zeros128 = jnp.zeros((128, 128), jnp.float32)
  x_pad = jnp.pad(x, ((0, N_PAD - N), (0, 0)))
  alpha1 = alpha.reshape(1)

  pdeg = _deg_kernel(dst_dg, ones128, zeros128).reshape(NC, N_PAD, 128)

  hh0, hs0 = _tc1(x_pad, W0, pdeg)
  agg0 = _segsum_kernel(hs0.reshape(4 * N_PAD, 128), src4, dst_ss,
                        zeros128, 4).reshape(4, N_PAD, 128)

  hh1, hs1, resid1 = _tc2(agg0, hh0, pdeg, b0.reshape(1, 512), alpha1,
                          W1, Ws0, bs0.reshape(1, 512))
  agg1 = _segsum_kernel(hs1.reshape(4 * N_PAD, 128), src4, dst_ss,
                        zeros128, 4).reshape(4, N_PAD, 128)

  hh2, hs2, resid2 = _tc3(agg1, hh1, resid1, pdeg, b1.reshape(1, 512),
                          W2, Ws1, bs1.reshape(1, 256))
  agg2 = _segsum_kernel(hs2.reshape(2 * N_PAD, 128), src2, dst_ss,
                        zeros128, 2).reshape(2, N_PAD, 128)

  out = _tc4(agg2, hh2, resid2, pdeg, b2.reshape(1, 256),
             Wp0, bp0.reshape(1, 256), Wp1, bp1.reshape(1, 128))
  return out[:N]
